# baseline (speedup 1.0000x reference)
import functools

import jax
import jax.numpy as jnp
from jax import lax
from jax.experimental import pallas as pl
from jax.experimental.pallas import tpu as pltpu
from jax.experimental.pallas import tpu_sc as plsc

N_NODES = 10000
N_EDGES = 320000
FEATS = 128
MSG_W = 32
GW = 128
N_SUBCORES = 16
ZROWS = N_NODES // N_SUBCORES

BN = 1000
PACK = 4
BEP = 1000

_f32 = jnp.float32
_bf16 = jnp.bfloat16


def _mesh():
    return plsc.VectorSubcoreMesh(core_axis_name="core", subcore_axis_name="subcore")


def _sc_gather(tdst, tsrc, ei, ea):
    grid = N_EDGES // GW
    srows = N_NODES // N_SUBCORES

    @functools.partial(
        pl.kernel,
        mesh=_mesh(),
        out_type=jax.ShapeDtypeStruct((N_EDGES, MSG_W), _f32),
        scratch_types=[
            pltpu.VMEM_SHARED((N_NODES, MSG_W), _f32),
            pltpu.VMEM_SHARED((N_NODES, MSG_W), _f32),
        ],
        compiler_params=pltpu.CompilerParams(use_tc_tiling_on_sc=False),
    )
    def k(td_hbm, ts_hbm, ei_hbm, ea_hbm, g_hbm, tabd, tabs):
        s = lax.axis_index("subcore")
        pltpu.sync_copy(td_hbm.at[pl.ds(s * srows, srows)],
                        tabd.at[pl.ds(s * srows, srows)])
        pltpu.sync_copy(ts_hbm.at[pl.ds(s * srows, srows)],
                        tabs.at[pl.ds(s * srows, srows)])
        plsc.subcore_barrier()

        def body(si_vmem, di_vmem, g_vmem):
            i = pl.program_id(0)
            pltpu.sync_copy(tabd.at[di_vmem.at[0]], g_vmem)
            pltpu.sync_copy(tabs.at[si_vmem.at[0]], g_vmem, add=True)
            pltpu.sync_copy(ea_hbm.at[pl.ds(i * GW, GW)],
                            g_vmem.at[:, pl.ds(19, 4)])

        pltpu.emit_pipeline(
            body,
            grid=(grid,),
            in_specs=[
                pl.BlockSpec((1, GW), lambda i: (0, i)),
                pl.BlockSpec((1, GW), lambda i: (1, i)),
            ],
            out_specs=[
                pl.BlockSpec((GW, MSG_W), lambda i: (i, 0)),
            ],
            core_axis_name=("core", "subcore"),
            dimension_semantics=(pltpu.PARALLEL,),
        )(ei_hbm, ei_hbm, g_hbm)

    return k(tdst, tsrc, ei, ea)


def _sc_scatter_add(msg, ei, zrows):
    grid = N_EDGES // GW

    @functools.partial(
        pl.kernel,
        mesh=_mesh(),
        out_type=jax.ShapeDtypeStruct((2, N_NODES, MSG_W), _f32),
        scratch_types=[pltpu.VMEM_SHARED((N_NODES, MSG_W), _f32)],
        compiler_params=pltpu.CompilerParams(use_tc_tiling_on_sc=False),
    )
    def k(msg_hbm, di_hbm, z_hbm, out_hbm, acc):
        c = lax.axis_index("core")
        s = lax.axis_index("subcore")
        pltpu.sync_copy(z_hbm, acc.at[pl.ds(s * ZROWS, ZROWS)])
        plsc.subcore_barrier()

        def body(m_vmem, di_vmem):
            pltpu.sync_copy(m_vmem, acc.at[di_vmem.at[0]], add=True)

        pltpu.emit_pipeline(
            body,
            grid=(grid,),
            in_specs=[
                pl.BlockSpec((GW, MSG_W), lambda i: (i, 0)),
                pl.BlockSpec((1, GW), lambda i: (1, i)),
            ],
            out_specs=[],
            core_axis_name=("core", "subcore"),
            dimension_semantics=(pltpu.PARALLEL,),
        )(msg_hbm, di_hbm)

        plsc.subcore_barrier()
        pltpu.sync_copy(
            acc.at[pl.ds(s * ZROWS, ZROWS)],
            out_hbm.at[c].at[pl.ds(s * ZROWS, ZROWS)],
        )

    return k(msg, ei, zrows)


def _tc_tables(h, pos, wd, ws):
    def body(h_ref, p_ref, wd_ref, ws_ref, td_ref, ts_ref):
        hb = h_ref[...]
        pb = p_ref[...]
        z = jnp.zeros((hb.shape[0], MSG_W - 19), _f32)
        a = jnp.dot(hb, wd_ref[...], preferred_element_type=_f32)
        b = jnp.dot(hb, ws_ref[...], preferred_element_type=_f32)
        td_ref[...] = jnp.concatenate([a, pb, z], axis=1)
        ts_ref[...] = jnp.concatenate([b, -pb, z], axis=1)

    grid = (N_NODES // BN,)
    return pl.pallas_call(
        body,
        grid=grid,
        in_specs=[
            pl.BlockSpec((BN, FEATS), lambda i: (i, 0)),
            pl.BlockSpec((BN, 3), lambda i: (i, 0)),
            pl.BlockSpec((FEATS, 16), lambda i: (0, 0)),
            pl.BlockSpec((FEATS, 16), lambda i: (0, 0)),
        ],
        out_specs=[
            pl.BlockSpec((BN, MSG_W), lambda i: (i, 0)),
            pl.BlockSpec((BN, MSG_W), lambda i: (i, 0)),
        ],
        out_shape=(
            jax.ShapeDtypeStruct((N_NODES, MSG_W), _f32),
            jax.ShapeDtypeStruct((N_NODES, MSG_W), _f32),
        ),
    )(h, pos, wd, ws)


def _tc_edge(gin, pw):
    def body(g_ref, w1a_ref, sp_ref, e2p_ref, x1p_ref,
             x2p_ref, b1_ref, b2_ref, bx1_ref, bx2_ref, out_ref):
        g = g_ref[...]
        pre = (jnp.dot(g.astype(_bf16), w1a_ref[...], preferred_element_type=_f32)
               + jnp.dot((g * g).astype(_bf16), sp_ref[...],
                         preferred_element_type=_f32)
               + b1_ref[...])
        m1 = jax.nn.silu(pre)
        m = jax.nn.silu(jnp.dot(m1.astype(_bf16), e2p_ref[...],
                                preferred_element_type=_f32) + b2_ref[...])
        t = jax.nn.silu(jnp.dot(m.astype(_bf16), x1p_ref[...],
                                preferred_element_type=_f32) + bx1_ref[...])
        xwb = (jnp.dot(t.astype(_bf16), x2p_ref[...],
                       preferred_element_type=_f32) + bx2_ref[...])
        out_ref[...] = m + g * xwb

    rows = N_EDGES // PACK
    grid = (rows // BEP,)
    full = lambda shp: pl.BlockSpec(shp, lambda i: (0, 0))
    out = pl.pallas_call(
        body,
        grid=grid,
        in_specs=[
            pl.BlockSpec((BEP, 128), lambda i: (i, 0)),
            full((128, 128)), full((128, 128)),
            full((128, 128)), full((128, 128)), full((128, 128)),
            full((1, 128)), full((1, 128)), full((1, 128)), full((1, 128)),
        ],
        out_specs=pl.BlockSpec((BEP, 128), lambda i: (i, 0)),
        out_shape=jax.ShapeDtypeStruct((rows, 128), _f32),
    )(gin,
      pw['w1a'], pw['sp'], pw['e2p'], pw['x1p'], pw['x2p'],
      pw['b1'], pw['b2'], pw['bx1'], pw['bx2'])
    return out


def _tc_update(h, pos, p, h1a, h1b, bh1, h2w, bh2, wd, ws):
    def body(h_ref, pos_ref, p_ref, h1a_ref, h1b_ref, bh1_ref, h2w_ref,
             bh2_ref, wd_ref, ws_ref, hn_ref, td_ref, ts_ref):
        hb = h_ref[...]
        pb = pos_ref[...]
        agg = p_ref[0] + p_ref[1]
        magg = agg[:, :16]
        pagg = agg[:, 16:19]
        hu = jax.nn.silu(
            jnp.dot(hb, h1a_ref[...], preferred_element_type=_f32)
            + jnp.dot(magg, h1b_ref[...], preferred_element_type=_f32)
            + bh1_ref[...])
        hu = jnp.dot(hu, h2w_ref[...], preferred_element_type=_f32) + bh2_ref[...]
        hn = hb + hu
        pn = pb + pagg
        hn_ref[...] = hn
        z = jnp.zeros((hb.shape[0], MSG_W - 19), _f32)
        a = jnp.dot(hn, wd_ref[...], preferred_element_type=_f32)
        b = jnp.dot(hn, ws_ref[...], preferred_element_type=_f32)
        td_ref[...] = jnp.concatenate([a, pn, z], axis=1)
        ts_ref[...] = jnp.concatenate([b, -pn, z], axis=1)

    grid = (N_NODES // BN,)
    full = lambda shp: pl.BlockSpec(shp, lambda i: (0, 0))
    return pl.pallas_call(
        body,
        grid=grid,
        in_specs=[
            pl.BlockSpec((BN, FEATS), lambda i: (i, 0)),
            pl.BlockSpec((BN, 3), lambda i: (i, 0)),
            pl.BlockSpec((2, BN, MSG_W), lambda i: (0, i, 0)),
            full((FEATS, FEATS)), full((16, FEATS)), full((1, FEATS)),
            full((FEATS, FEATS)), full((1, FEATS)),
            full((FEATS, 16)), full((FEATS, 16)),
        ],
        out_specs=[
            pl.BlockSpec((BN, FEATS), lambda i: (i, 0)),
            pl.BlockSpec((BN, MSG_W), lambda i: (i, 0)),
            pl.BlockSpec((BN, MSG_W), lambda i: (i, 0)),
        ],
        out_shape=(
            jax.ShapeDtypeStruct((N_NODES, FEATS), _f32),
            jax.ShapeDtypeStruct((N_NODES, MSG_W), _f32),
            jax.ShapeDtypeStruct((N_NODES, MSG_W), _f32),
        ),
    )(h, pos, p, h1a, h1b, bh1, h2w, bh2, wd, ws)


def _tc_final(h, p, h1a, h1b, bh1, h2w, bh2, linw, linb):
    def body(h_ref, p_ref, h1a_ref, h1b_ref, bh1_ref, h2w_ref, bh2_ref,
             linw_ref, linb_ref, out_ref):
        hb = h_ref[...]
        agg = p_ref[0] + p_ref[1]
        magg = agg[:, :16]
        hu = jax.nn.silu(
            jnp.dot(hb, h1a_ref[...], preferred_element_type=_f32)
            + jnp.dot(magg, h1b_ref[...], preferred_element_type=_f32)
            + bh1_ref[...])
        hu = jnp.dot(hu, h2w_ref[...], preferred_element_type=_f32) + bh2_ref[...]
        hn = hb + hu
        out_ref[...] = (jnp.dot(hn, linw_ref[...], preferred_element_type=_f32)
                        + linb_ref[...])

    grid = (N_NODES // BN,)
    full = lambda shp: pl.BlockSpec(shp, lambda i: (0, 0))
    return pl.pallas_call(
        body,
        grid=grid,
        in_specs=[
            pl.BlockSpec((BN, FEATS), lambda i: (i, 0)),
            pl.BlockSpec((2, BN, MSG_W), lambda i: (0, i, 0)),
            full((FEATS, FEATS)), full((16, FEATS)), full((1, FEATS)),
            full((FEATS, FEATS)), full((1, FEATS)),
            full((FEATS, 16)), full((1, 16)),
        ],
        out_specs=pl.BlockSpec((BN, 16), lambda i: (i, 0)),
        out_shape=jax.ShapeDtypeStruct((N_NODES, 16), _f32),
    )(h, p, h1a, h1b, bh1, h2w, bh2, linw, linb)


def _layer_weights(p):
    w1, b1 = p['e1']
    return dict(
        wd=w1[:FEATS], ws=w1[FEATS:2 * FEATS],
        wd2=w1[2 * FEATS:2 * FEATS + 1], wea=w1[2 * FEATS + 1:],
        be1=b1.reshape(1, -1),
        e2w=p['e2'][0], be2=p['e2'][1].reshape(1, -1),
        x1w=p['x1'][0], bx1=p['x1'][1].reshape(1, -1),
        x2w=p['x2'][0], bx2=p['x2'][1].reshape(1, -1),
        h1a=p['h1'][0][:FEATS], h1b=p['h1'][0][FEATS:],
        bh1=p['h1'][1].reshape(1, -1),
        h2w=p['h2'][0], bh2=p['h2'][1].reshape(1, -1),
    )


def _embed(w, r0, c0, rr, cc):
    h, wd = w.shape
    row = [w]
    if c0:
        row.insert(0, jnp.zeros((h, c0), _f32))
    if cc - c0 - wd:
        row.append(jnp.zeros((h, cc - c0 - wd), _f32))
    mid = jnp.concatenate(row, axis=1) if len(row) > 1 else w
    col = [mid]
    if r0:
        col.insert(0, jnp.zeros((r0, cc), _f32))
    if rr - r0 - h:
        col.append(jnp.zeros((rr - r0 - h, cc), _f32))
    return jnp.concatenate(col, axis=0) if len(col) > 1 else mid


def _tc_prep(l0, l1):
    names = ('wd2', 'wea', 'e2w', 'x1w', 'x2w', 'be1', 'be2', 'bx1', 'bx2')
    ins = [l0[n] for n in names] + [l1[n] for n in names]

    def one(wd2, wea, e2w, x1w, x2w, be1, be2, bx1, bx2):
        wd2b = jnp.broadcast_to(wd2, (3, 16))
        x2b = jnp.broadcast_to(x2w, (16, 3))
        bx2b = jnp.broadcast_to(bx2, (1, 3))
        r = lax.broadcasted_iota(jnp.int32, (128, 128), 0)
        c = lax.broadcasted_iota(jnp.int32, (128, 128), 1)
        eye = jnp.where(r == c, jnp.float32(1), jnp.float32(0))
        w1a = eye + sum(_embed(wea, 32 * i + 19, 32 * i, 128, 128)
                        for i in range(PACK))
        sp = sum(_embed(wd2b, 32 * i + 16, 32 * i, 128, 128) for i in range(PACK))
        e2p = sum(_embed(e2w, 32 * i, 32 * i, 128, 128) for i in range(PACK))
        x1p = sum(_embed(x1w, 32 * i, 32 * i, 128, 128) for i in range(PACK))
        x2p = sum(_embed(x2b, 32 * i, 32 * i + 16, 128, 128) for i in range(PACK))
        b1 = sum(_embed(be1, 0, 32 * i, 1, 128) for i in range(PACK))
        b2 = sum(_embed(be2, 0, 32 * i, 1, 128) for i in range(PACK))
        bx1v = sum(_embed(bx1, 0, 32 * i, 1, 128) for i in range(PACK))
        bx2v = sum(_embed(bx2b, 0, 32 * i + 16, 1, 128) for i in range(PACK))
        return w1a, sp, e2p, x1p, x2p, b1, b2, bx1v, bx2v

    def body(*refs):
        in_refs = refs[:len(ins)]
        out_refs = refs[len(ins):]
        vals = [r[...] for r in in_refs]
        outs = one(*vals[:9]) + one(*vals[9:])
        for r, v in zip(out_refs, outs):
            r[...] = v.astype(r.dtype)

    full = lambda a: pl.BlockSpec(a.shape, lambda: tuple(0 for _ in a.shape))
    oshapes = []
    for _ in range(2):
        oshapes += [((128, 128), _bf16), ((128, 128), _bf16),
                    ((128, 128), _bf16), ((128, 128), _bf16),
                    ((128, 128), _bf16),
                    ((1, 128), _f32), ((1, 128), _f32),
                    ((1, 128), _f32), ((1, 128), _f32)]
    outs = pl.pallas_call(
        body,
        in_specs=[full(a) for a in ins],
        out_specs=[pl.BlockSpec(s, lambda: tuple(0 for _ in s))
                   for s, _ in oshapes],
        out_shape=[jax.ShapeDtypeStruct(s, d) for s, d in oshapes],
    )(*ins)
    keys = ('w1a', 'sp', 'e2p', 'x1p', 'x2p', 'b1', 'b2', 'bx1', 'bx2')
    pw0 = dict(zip(keys, outs[:9]))
    pw1 = dict(zip(keys, outs[9:]))
    return pw0, pw1


def kernel(x, edge_index, edge_attr, pos, params):
    rows = N_EDGES // PACK
    zrows = jnp.zeros((ZROWS, MSG_W), _f32)

    l0 = _layer_weights(params['layers'][0])
    l1 = _layer_weights(params['layers'][1])
    pw0, pw1 = _tc_prep(l0, l1)
    linw, linb = params['lin1']
    linb = linb.reshape(1, -1)

    td, ts = _tc_tables(x, pos, l0['wd'], l0['ws'])
    g = _sc_gather(td, ts, edge_index, edge_attr)
    msg = _tc_edge(g.reshape(rows, 128), pw0)
    p0 = _sc_scatter_add(msg.reshape(N_EDGES, MSG_W), edge_index, zrows)

    h1, td2, ts2 = _tc_update(x, pos, p0, l0['h1a'], l0['h1b'], l0['bh1'],
                              l0['h2w'], l0['bh2'], l1['wd'], l1['ws'])

    g2 = _sc_gather(td2, ts2, edge_index, edge_attr)
    msg2 = _tc_edge(g2.reshape(rows, 128), pw1)
    p1 = _sc_scatter_add(msg2.reshape(N_EDGES, MSG_W), edge_index, zrows)

    return _tc_final(h1, p1, l1['h1a'], l1['h1b'], l1['bh1'],
                     l1['h2w'], l1['bh2'], linw, linb)


# speedup vs baseline: 1.4435x; 1.4435x over previous
import functools

import jax
import jax.numpy as jnp
from jax import lax
from jax.experimental import pallas as pl
from jax.experimental.pallas import tpu as pltpu
from jax.experimental.pallas import tpu_sc as plsc

N_NODES = 10000
N_EDGES = 320000
FEATS = 128
MSG_W = 32
GW = 128
N_SUBCORES = 16
ZROWS = N_NODES // N_SUBCORES

BN = 1000
PACK = 4
BEP = 1000

_f32 = jnp.float32


def _mesh():
    return plsc.VectorSubcoreMesh(core_axis_name="core", subcore_axis_name="subcore")


def _sc_gather(tdst, tsrc, ei):
    grid = N_EDGES // GW
    srows = N_NODES // N_SUBCORES

    @functools.partial(
        pl.kernel,
        mesh=_mesh(),
        out_type=jax.ShapeDtypeStruct((N_EDGES, MSG_W), _f32),
        scratch_types=[
            pltpu.VMEM_SHARED((N_NODES, MSG_W), _f32),
            pltpu.VMEM_SHARED((N_NODES, MSG_W), _f32),
        ],
        compiler_params=pltpu.CompilerParams(use_tc_tiling_on_sc=False),
    )
    def k(td_hbm, ts_hbm, ei_hbm, g_hbm, tabd, tabs):
        s = lax.axis_index("subcore")
        pltpu.sync_copy(td_hbm.at[pl.ds(s * srows, srows)],
                        tabd.at[pl.ds(s * srows, srows)])
        pltpu.sync_copy(ts_hbm.at[pl.ds(s * srows, srows)],
                        tabs.at[pl.ds(s * srows, srows)])
        plsc.subcore_barrier()

        def body(si_vmem, di_vmem, g_vmem):
            pltpu.sync_copy(tabd.at[di_vmem.at[0]], g_vmem)
            pltpu.sync_copy(tabs.at[si_vmem.at[0]], g_vmem, add=True)

        pltpu.emit_pipeline(
            body,
            grid=(grid,),
            in_specs=[
                pl.BlockSpec((1, GW), lambda i: (0, i)),
                pl.BlockSpec((1, GW), lambda i: (1, i)),
            ],
            out_specs=[
                pl.BlockSpec((GW, MSG_W), lambda i: (i, 0)),
            ],
            core_axis_name=("core", "subcore"),
            dimension_semantics=(pltpu.PARALLEL,),
        )(ei_hbm, ei_hbm, g_hbm)

    return k(tdst, tsrc, ei)


def _sc_scatter_add(msg, ei, zrows):
    grid = N_EDGES // GW

    @functools.partial(
        pl.kernel,
        mesh=_mesh(),
        out_type=jax.ShapeDtypeStruct((2, N_NODES, MSG_W), _f32),
        scratch_types=[pltpu.VMEM_SHARED((N_NODES, MSG_W), _f32)],
        compiler_params=pltpu.CompilerParams(use_tc_tiling_on_sc=False),
    )
    def k(msg_hbm, di_hbm, z_hbm, out_hbm, acc):
        c = lax.axis_index("core")
        s = lax.axis_index("subcore")
        pltpu.sync_copy(z_hbm, acc.at[pl.ds(s * ZROWS, ZROWS)])
        plsc.subcore_barrier()

        def body(m_vmem, di_vmem):
            pltpu.sync_copy(m_vmem, acc.at[di_vmem.at[0]], add=True)

        pltpu.emit_pipeline(
            body,
            grid=(grid,),
            in_specs=[
                pl.BlockSpec((GW, MSG_W), lambda i: (i, 0)),
                pl.BlockSpec((1, GW), lambda i: (1, i)),
            ],
            out_specs=[],
            core_axis_name=("core", "subcore"),
            dimension_semantics=(pltpu.PARALLEL,),
        )(msg_hbm, di_hbm)

        plsc.subcore_barrier()
        pltpu.sync_copy(
            acc.at[pl.ds(s * ZROWS, ZROWS)],
            out_hbm.at[c].at[pl.ds(s * ZROWS, ZROWS)],
        )

    return k(msg, ei, zrows)


def _tc_tables(h, pos, wd, ws):
    def body(h_ref, p_ref, wd_ref, ws_ref, td_ref, ts_ref):
        hb = h_ref[...]
        pb = p_ref[...]
        z = jnp.zeros((hb.shape[0], MSG_W - 19), _f32)
        a = jnp.dot(hb, wd_ref[...], preferred_element_type=_f32)
        b = jnp.dot(hb, ws_ref[...], preferred_element_type=_f32)
        td_ref[...] = jnp.concatenate([a, pb, z], axis=1)
        ts_ref[...] = jnp.concatenate([b, -pb, z], axis=1)

    grid = (N_NODES // BN,)
    return pl.pallas_call(
        body,
        grid=grid,
        in_specs=[
            pl.BlockSpec((BN, FEATS), lambda i: (i, 0)),
            pl.BlockSpec((BN, 3), lambda i: (i, 0)),
            pl.BlockSpec((FEATS, 16), lambda i: (0, 0)),
            pl.BlockSpec((FEATS, 16), lambda i: (0, 0)),
        ],
        out_specs=[
            pl.BlockSpec((BN, MSG_W), lambda i: (i, 0)),
            pl.BlockSpec((BN, MSG_W), lambda i: (i, 0)),
        ],
        out_shape=(
            jax.ShapeDtypeStruct((N_NODES, MSG_W), _f32),
            jax.ShapeDtypeStruct((N_NODES, MSG_W), _f32),
        ),
    )(h, pos, wd, ws)


def _tc_edge(gin, eap, pw):
    def body(g_ref, ea_ref, eaw_ref, sp_ref, e2p_ref, x1p_ref,
             x2p_ref, b1_ref, b2_ref, bx1_ref, bx2_ref, out_ref):
        g = g_ref[...]
        pre = (g
               + jnp.dot(ea_ref[...], eaw_ref[...], preferred_element_type=_f32)
               + jnp.dot(g * g, sp_ref[...], preferred_element_type=_f32)
               + b1_ref[...])
        m1 = jax.nn.silu(pre)
        m = jax.nn.silu(jnp.dot(m1, e2p_ref[...], preferred_element_type=_f32)
                        + b2_ref[...])
        t = jax.nn.silu(jnp.dot(m, x1p_ref[...], preferred_element_type=_f32)
                        + bx1_ref[...])
        xwb = jnp.dot(t, x2p_ref[...], preferred_element_type=_f32) + bx2_ref[...]
        out_ref[...] = m + g * xwb

    rows = N_EDGES // PACK
    grid = (rows // BEP,)
    full = lambda shp: pl.BlockSpec(shp, lambda i: (0, 0))
    out = pl.pallas_call(
        body,
        grid=grid,
        in_specs=[
            pl.BlockSpec((BEP, 128), lambda i: (i, 0)),
            pl.BlockSpec((BEP, 16), lambda i: (i, 0)),
            full((16, 128)), full((128, 128)),
            full((128, 128)), full((128, 128)), full((128, 128)),
            full((1, 128)), full((1, 128)), full((1, 128)), full((1, 128)),
        ],
        out_specs=pl.BlockSpec((BEP, 128), lambda i: (i, 0)),
        out_shape=jax.ShapeDtypeStruct((rows, 128), _f32),
    )(gin, eap,
      pw['eaw'], pw['sp'], pw['e2p'], pw['x1p'], pw['x2p'],
      pw['b1'], pw['b2'], pw['bx1'], pw['bx2'])
    return out


def _tc_update(h, pos, p, h1a, h1b, bh1, h2w, bh2, wd, ws):
    def body(h_ref, pos_ref, p_ref, h1a_ref, h1b_ref, bh1_ref, h2w_ref,
             bh2_ref, wd_ref, ws_ref, hn_ref, td_ref, ts_ref):
        hb = h_ref[...]
        pb = pos_ref[...]
        agg = p_ref[0] + p_ref[1]
        magg = agg[:, :16]
        pagg = agg[:, 16:19]
        hu = jax.nn.silu(
            jnp.dot(hb, h1a_ref[...], preferred_element_type=_f32)
            + jnp.dot(magg, h1b_ref[...], preferred_element_type=_f32)
            + bh1_ref[...])
        hu = jnp.dot(hu, h2w_ref[...], preferred_element_type=_f32) + bh2_ref[...]
        hn = hb + hu
        pn = pb + pagg
        hn_ref[...] = hn
        z = jnp.zeros((hb.shape[0], MSG_W - 19), _f32)
        a = jnp.dot(hn, wd_ref[...], preferred_element_type=_f32)
        b = jnp.dot(hn, ws_ref[...], preferred_element_type=_f32)
        td_ref[...] = jnp.concatenate([a, pn, z], axis=1)
        ts_ref[...] = jnp.concatenate([b, -pn, z], axis=1)

    grid = (N_NODES // BN,)
    full = lambda shp: pl.BlockSpec(shp, lambda i: (0, 0))
    return pl.pallas_call(
        body,
        grid=grid,
        in_specs=[
            pl.BlockSpec((BN, FEATS), lambda i: (i, 0)),
            pl.BlockSpec((BN, 3), lambda i: (i, 0)),
            pl.BlockSpec((2, BN, MSG_W), lambda i: (0, i, 0)),
            full((FEATS, FEATS)), full((16, FEATS)), full((1, FEATS)),
            full((FEATS, FEATS)), full((1, FEATS)),
            full((FEATS, 16)), full((FEATS, 16)),
        ],
        out_specs=[
            pl.BlockSpec((BN, FEATS), lambda i: (i, 0)),
            pl.BlockSpec((BN, MSG_W), lambda i: (i, 0)),
            pl.BlockSpec((BN, MSG_W), lambda i: (i, 0)),
        ],
        out_shape=(
            jax.ShapeDtypeStruct((N_NODES, FEATS), _f32),
            jax.ShapeDtypeStruct((N_NODES, MSG_W), _f32),
            jax.ShapeDtypeStruct((N_NODES, MSG_W), _f32),
        ),
    )(h, pos, p, h1a, h1b, bh1, h2w, bh2, wd, ws)


def _tc_final(h, p, h1a, h1b, bh1, h2w, bh2, linw, linb):
    def body(h_ref, p_ref, h1a_ref, h1b_ref, bh1_ref, h2w_ref, bh2_ref,
             linw_ref, linb_ref, out_ref):
        hb = h_ref[...]
        agg = p_ref[0] + p_ref[1]
        magg = agg[:, :16]
        hu = jax.nn.silu(
            jnp.dot(hb, h1a_ref[...], preferred_element_type=_f32)
            + jnp.dot(magg, h1b_ref[...], preferred_element_type=_f32)
            + bh1_ref[...])
        hu = jnp.dot(hu, h2w_ref[...], preferred_element_type=_f32) + bh2_ref[...]
        hn = hb + hu
        out_ref[...] = (jnp.dot(hn, linw_ref[...], preferred_element_type=_f32)
                        + linb_ref[...])

    grid = (N_NODES // BN,)
    full = lambda shp: pl.BlockSpec(shp, lambda i: (0, 0))
    return pl.pallas_call(
        body,
        grid=grid,
        in_specs=[
            pl.BlockSpec((BN, FEATS), lambda i: (i, 0)),
            pl.BlockSpec((2, BN, MSG_W), lambda i: (0, i, 0)),
            full((FEATS, FEATS)), full((16, FEATS)), full((1, FEATS)),
            full((FEATS, FEATS)), full((1, FEATS)),
            full((FEATS, 16)), full((1, 16)),
        ],
        out_specs=pl.BlockSpec((BN, 16), lambda i: (i, 0)),
        out_shape=jax.ShapeDtypeStruct((N_NODES, 16), _f32),
    )(h, p, h1a, h1b, bh1, h2w, bh2, linw, linb)


def _layer_weights(p):
    w1, b1 = p['e1']
    return dict(
        wd=w1[:FEATS], ws=w1[FEATS:2 * FEATS],
        wd2=w1[2 * FEATS:2 * FEATS + 1], wea=w1[2 * FEATS + 1:],
        be1=b1.reshape(1, -1),
        e2w=p['e2'][0], be2=p['e2'][1].reshape(1, -1),
        x1w=p['x1'][0], bx1=p['x1'][1].reshape(1, -1),
        x2w=p['x2'][0], bx2=p['x2'][1].reshape(1, -1),
        h1a=p['h1'][0][:FEATS], h1b=p['h1'][0][FEATS:],
        bh1=p['h1'][1].reshape(1, -1),
        h2w=p['h2'][0], bh2=p['h2'][1].reshape(1, -1),
    )


def _embed(w, r0, c0, rr, cc):
    h, wd = w.shape
    row = [w]
    if c0:
        row.insert(0, jnp.zeros((h, c0), _f32))
    if cc - c0 - wd:
        row.append(jnp.zeros((h, cc - c0 - wd), _f32))
    mid = jnp.concatenate(row, axis=1) if len(row) > 1 else w
    col = [mid]
    if r0:
        col.insert(0, jnp.zeros((r0, cc), _f32))
    if rr - r0 - h:
        col.append(jnp.zeros((rr - r0 - h, cc), _f32))
    return jnp.concatenate(col, axis=0) if len(col) > 1 else mid


def _tc_prep(l0, l1):
    names = ('wd2', 'wea', 'e2w', 'x1w', 'x2w', 'be1', 'be2', 'bx1', 'bx2')
    ins = [l0[n] for n in names] + [l1[n] for n in names]

    def one(wd2, wea, e2w, x1w, x2w, be1, be2, bx1, bx2):
        wd2b = jnp.broadcast_to(wd2, (3, 16))
        x2b = jnp.broadcast_to(x2w, (16, 3))
        bx2b = jnp.broadcast_to(bx2, (1, 3))
        eaw = sum(_embed(wea, 4 * i, 32 * i, 16, 128) for i in range(PACK))
        sp = sum(_embed(wd2b, 32 * i + 16, 32 * i, 128, 128) for i in range(PACK))
        e2p = sum(_embed(e2w, 32 * i, 32 * i, 128, 128) for i in range(PACK))
        x1p = sum(_embed(x1w, 32 * i, 32 * i, 128, 128) for i in range(PACK))
        x2p = sum(_embed(x2b, 32 * i, 32 * i + 16, 128, 128) for i in range(PACK))
        b1 = sum(_embed(be1, 0, 32 * i, 1, 128) for i in range(PACK))
        b2 = sum(_embed(be2, 0, 32 * i, 1, 128) for i in range(PACK))
        bx1v = sum(_embed(bx1, 0, 32 * i, 1, 128) for i in range(PACK))
        bx2v = sum(_embed(bx2b, 0, 32 * i + 16, 1, 128) for i in range(PACK))
        return eaw, sp, e2p, x1p, x2p, b1, b2, bx1v, bx2v

    def body(*refs):
        in_refs = refs[:len(ins)]
        out_refs = refs[len(ins):]
        vals = [r[...] for r in in_refs]
        outs = one(*vals[:9]) + one(*vals[9:])
        for r, v in zip(out_refs, outs):
            r[...] = v

    full = lambda a: pl.BlockSpec(a.shape, lambda: tuple(0 for _ in a.shape))
    oshapes = []
    for _ in range(2):
        oshapes += [(16, 128), (128, 128), (128, 128), (128, 128), (128, 128),
                    (1, 128), (1, 128), (1, 128), (1, 128)]
    outs = pl.pallas_call(
        body,
        in_specs=[full(a) for a in ins],
        out_specs=[pl.BlockSpec(s, lambda: tuple(0 for _ in s)) for s in oshapes],
        out_shape=[jax.ShapeDtypeStruct(s, _f32) for s in oshapes],
    )(*ins)
    keys = ('eaw', 'sp', 'e2p', 'x1p', 'x2p', 'b1', 'b2', 'bx1', 'bx2')
    pw0 = dict(zip(keys, outs[:9]))
    pw1 = dict(zip(keys, outs[9:]))
    return pw0, pw1


def kernel(x, edge_index, edge_attr, pos, params):
    rows = N_EDGES // PACK
    zrows = jnp.zeros((ZROWS, MSG_W), _f32)
    ea_p = edge_attr.reshape(rows, PACK * 4)

    l0 = _layer_weights(params['layers'][0])
    l1 = _layer_weights(params['layers'][1])
    pw0, pw1 = _tc_prep(l0, l1)
    linw, linb = params['lin1']
    linb = linb.reshape(1, -1)

    td, ts = _tc_tables(x, pos, l0['wd'], l0['ws'])
    g = _sc_gather(td, ts, edge_index)
    msg = _tc_edge(g.reshape(rows, 128), ea_p, pw0)
    p0 = _sc_scatter_add(msg.reshape(N_EDGES, MSG_W), edge_index, zrows)

    h1, td2, ts2 = _tc_update(x, pos, p0, l0['h1a'], l0['h1b'], l0['bh1'],
                              l0['h2w'], l0['bh2'], l1['wd'], l1['ws'])

    g2 = _sc_gather(td2, ts2, edge_index)
    msg2 = _tc_edge(g2.reshape(rows, 128), ea_p, pw1)
    p1 = _sc_scatter_add(msg2.reshape(N_EDGES, MSG_W), edge_index, zrows)

    return _tc_final(h1, p1, l1['h1a'], l1['h1b'], l1['bh1'],
                     l1['h2w'], l1['bh2'], linw, linb)
